# baseline (speedup 1.0000x reference)
import functools

import jax
import jax.numpy as jnp
from jax import lax
from jax.experimental import pallas as pl
from jax.experimental.pallas import tpu as pltpu
from jax.experimental.pallas import tpu_sc as plsc

N = 10000
NE = 10000
NNZ = 320000
DIN = 128
DIM = 64
NC = 40

NCORES = 2
NSUB = 16
NPAD = 10112
ROWS_PER_TILE = NPAD // NSUB
IDX_BLK = 128
DCHUNK = ROWS_PER_TILE // 4
NBLK = 2560
BLK_PER_TILE = NBLK // NSUB
NNZ_PAD = NBLK * IDX_BLK - NNZ

_sc_mesh = plsc.VectorSubcoreMesh(
    core_axis_name="c", subcore_axis_name="s",
    num_cores=NCORES, num_subcores=NSUB)
_sc_params = pltpu.CompilerParams(use_tc_tiling_on_sc=False)


def _degree_body(ni_hbm, ei_hbm, dinv_hbm, binv_hbm,
                 iv, ones_v, dbuf, acc_sh):
    c = lax.axis_index("c")
    s = lax.axis_index("s")
    base = s * ROWS_PER_TILE
    t0 = s * BLK_PER_TILE

    @pl.when(c == 0)
    def _():
        pltpu.sync_copy(ni_hbm.at[pl.ds(t0, BLK_PER_TILE)], iv)

    @pl.when(c == 1)
    def _():
        pltpu.sync_copy(ei_hbm.at[pl.ds(t0, BLK_PER_TILE)], iv)

    def fill(r, carry):
        ones_v[r, :] = jnp.ones((16,), jnp.float32)
        dbuf[r, :] = jnp.zeros((16,), jnp.float32)
        return carry

    lax.fori_loop(0, IDX_BLK, fill, 0)

    def zero(r, carry):
        dbuf[r, :] = jnp.zeros((16,), jnp.float32)
        return carry

    lax.fori_loop(IDX_BLK, ROWS_PER_TILE, zero, 0)
    pltpu.sync_copy(dbuf, acc_sh.at[pl.ds(base, ROWS_PER_TILE)])
    plsc.subcore_barrier()

    def step(b, carry):
        pltpu.sync_copy(ones_v, acc_sh.at[iv.at[b]], add=True)
        return carry

    lax.fori_loop(0, BLK_PER_TILE, step, 0)
    plsc.subcore_barrier()

    pltpu.sync_copy(acc_sh.at[pl.ds(base, ROWS_PER_TILE)], dbuf)

    def inv(r, carry):
        v = dbuf[r, :]
        dbuf[r, :] = jnp.where(v > 0.0, 1.0 / v, 0.0)
        return carry

    lax.fori_loop(0, ROWS_PER_TILE, inv, 0)

    @pl.when(c == 0)
    def _():
        pltpu.sync_copy(dbuf, dinv_hbm.at[pl.ds(base, ROWS_PER_TILE)])

    @pl.when(c == 1)
    def _():
        pltpu.sync_copy(dbuf, binv_hbm.at[pl.ds(base, ROWS_PER_TILE)])


_degree = pl.kernel(
    _degree_body,
    out_type=(jax.ShapeDtypeStruct((NPAD, 16), jnp.float32),
              jax.ShapeDtypeStruct((NPAD, 16), jnp.float32)),
    mesh=_sc_mesh,
    scratch_types=[
        pltpu.VMEM((BLK_PER_TILE, IDX_BLK), jnp.int32),
        pltpu.VMEM((IDX_BLK, 16), jnp.float32),
        pltpu.VMEM((ROWS_PER_TILE, 16), jnp.float32),
        pltpu.VMEM_SHARED((NPAD, 16), jnp.float32),
    ],
    compiler_params=_sc_params,
)


NSLOT = 8
GRP = 8
NGRP = BLK_PER_TILE // GRP
_DRAIN_CHUNKS = [(0, 128), (128, 128), (256, 128), (384, 128), (512, 120)]


def _layer_body(F, xa, xb, ni_hbm, ei_hbm, binv_hbm, dinv_hbm,
                ea, eb, pa, pb,
                wg, ws, rows, sbuf, acc_sh, gsem, ssem, isem):
    c = lax.axis_index("c")
    s = lax.axis_index("s")
    base = s * ROWS_PER_TILE
    t0 = s * BLK_PER_TILE
    nk = F // 16

    def zero_acc():
        def zero(r, carry):
            for k in range(nk):
                rows[0, r, pl.ds(k * 16, 16)] = jnp.zeros((16,), jnp.float32)
            return carry

        lax.fori_loop(0, 128, zero, 0)
        for off, sz in _DRAIN_CHUNKS:
            pltpu.sync_copy(rows.at[0, pl.ds(0, sz)],
                            acc_sh.at[pl.ds(base + off, sz)])

    def accumulate(src_a, src_b, gsrc_hbm, ssrc_hbm):
        def prefetch(w, wslot):
            @pl.when(w < NGRP)
            def _():
                pltpu.async_copy(gsrc_hbm.at[pl.ds(t0 + w * GRP, GRP)],
                                 wg.at[wslot], isem)
                pltpu.async_copy(ssrc_hbm.at[pl.ds(t0 + w * GRP, GRP)],
                                 ws.at[wslot], isem)

        def wait_window(w, wslot):
            pltpu.make_async_copy(gsrc_hbm.at[pl.ds(t0 + w * GRP, GRP)],
                                  wg.at[wslot], isem).wait()
            pltpu.make_async_copy(ssrc_hbm.at[pl.ds(t0 + w * GRP, GRP)],
                                  ws.at[wslot], isem).wait()

        def gather(wslot, j, k):
            @pl.when(c == 0)
            def _():
                pltpu.async_copy(src_a.at[wg.at[wslot, j]], rows.at[k], gsem)

            @pl.when(c == 1)
            def _():
                pltpu.async_copy(src_b.at[wg.at[wslot, j]], rows.at[k], gsem)

        def wait_gather(wslot, j, k):
            pltpu.make_async_copy(src_a.at[wg.at[wslot, j]],
                                  rows.at[k], gsem).wait()

        def scatter(wslot, j, k):
            return pltpu.async_copy(rows.at[k], acc_sh.at[ws.at[wslot, j]],
                                    ssem, add=True)

        pltpu.sync_copy(gsrc_hbm.at[pl.ds(t0, GRP)], wg.at[0])
        pltpu.sync_copy(ssrc_hbm.at[pl.ds(t0, GRP)], ws.at[0])
        prefetch(1, 1)
        for k in range(NSLOT):
            gather(0, k, k)

        def group(g, carry):
            wcur = lax.rem(g, 3)
            wnxt = lax.rem(g + 1, 3)
            prefetch(g + 2, lax.rem(g + 2, 3))
            descs = []
            for bank in range(NSLOT // 2):
                k0 = 2 * bank
                wait_gather(wcur, k0, k0)
                wait_gather(wcur, k0 + 1, k0 + 1)
                descs.append(scatter(wcur, k0, k0))
                descs.append(scatter(wcur, k0 + 1, k0 + 1))
            for d in descs:
                d.wait()

            @pl.when(g + 1 < NGRP)
            def _():
                wait_window(g + 1, wnxt)
                for k in range(NSLOT):
                    gather(wnxt, k, k)
            return carry

        lax.fori_loop(0, NGRP, group, 0)

    def drain(scale_hbm, dst_a, dst_b):
        for off, sz in _DRAIN_CHUNKS:
            pltpu.sync_copy(scale_hbm.at[pl.ds(base + off, sz)],
                            sbuf.at[pl.ds(0, sz)])
            pltpu.sync_copy(acc_sh.at[pl.ds(base + off, sz)],
                            rows.at[0, pl.ds(0, sz)])

            def scale(r, carry):
                sv = sbuf[r, :]
                for k in range(nk):
                    rows[0, r, pl.ds(k * 16, 16)] = (
                        rows[0, r, pl.ds(k * 16, 16)] * sv)
                return carry

            lax.fori_loop(0, sz, scale, 0)

            @pl.when(c == 0)
            def _():
                pltpu.sync_copy(rows.at[0, pl.ds(0, sz)],
                                dst_a.at[pl.ds(base + off, sz)])

            @pl.when(c == 1)
            def _():
                pltpu.sync_copy(rows.at[0, pl.ds(0, sz)],
                                dst_b.at[pl.ds(base + off, sz)])

    zero_acc()
    plsc.subcore_barrier()
    accumulate(xa, xb, ni_hbm, ei_hbm)
    plsc.subcore_barrier()
    drain(binv_hbm, ea, eb)
    zero_acc()
    plsc.subcore_barrier()
    accumulate(ea, eb, ei_hbm, ni_hbm)
    plsc.subcore_barrier()
    drain(dinv_hbm, pa, pb)


def _make_layer(F):
    return pl.kernel(
        functools.partial(_layer_body, F),
        out_type=(jax.ShapeDtypeStruct((NPAD, F), jnp.float32),) * 4,
        mesh=_sc_mesh,
        scratch_types=[
            pltpu.VMEM((3, GRP, IDX_BLK), jnp.int32),
            pltpu.VMEM((3, GRP, IDX_BLK), jnp.int32),
            pltpu.VMEM((NSLOT, IDX_BLK, F), jnp.float32),
            pltpu.VMEM((IDX_BLK, 16), jnp.float32),
            pltpu.VMEM_SHARED((NPAD, F), jnp.float32),
            pltpu.SemaphoreType.DMA,
            pltpu.SemaphoreType.DMA,
            pltpu.SemaphoreType.DMA,
        ],
        compiler_params=_sc_params,
    )


_layer64 = _make_layer(64)
_layer48 = _make_layer(48)


_RB = 1264
_GRID = NPAD // _RB


def _mm1_body(x_ref, wc_ref, wh_ref, oa_ref, ob_ref):
    xb = x_ref[...]
    oa_ref[...] = jnp.dot(xb, wc_ref[...], preferred_element_type=jnp.float32)
    ob_ref[...] = jnp.dot(xb, wh_ref[...], preferred_element_type=jnp.float32)


_mm1 = pl.pallas_call(
    _mm1_body,
    grid=(_GRID,),
    in_specs=[
        pl.BlockSpec((_RB, DIN), lambda i: (i, 0)),
        pl.BlockSpec((DIN, DIM), lambda i: (0, 0)),
        pl.BlockSpec((DIN, DIM), lambda i: (0, 0)),
    ],
    out_specs=[
        pl.BlockSpec((_RB, DIM), lambda i: (i, 0)),
        pl.BlockSpec((_RB, DIM), lambda i: (i, 0)),
    ],
    out_shape=[
        jax.ShapeDtypeStruct((NPAD, DIM), jnp.float32),
        jax.ShapeDtypeStruct((NPAD, DIM), jnp.float32),
    ],
)


def _mid_body(pa_ref, pb_ref, b1c_ref, b1h_ref, wc_ref, wh_ref,
              qa_ref, qb_ref):
    ga = jax.nn.relu(pa_ref[...] + b1c_ref[...])
    gb = jax.nn.relu(pb_ref[...] + b1h_ref[...])
    qa_ref[...] = jnp.dot(ga, wc_ref[...], preferred_element_type=jnp.float32)
    qb_ref[...] = jnp.dot(gb, wh_ref[...], preferred_element_type=jnp.float32)


_mid = pl.pallas_call(
    _mid_body,
    grid=(_GRID,),
    in_specs=[
        pl.BlockSpec((_RB, DIM), lambda i: (i, 0)),
        pl.BlockSpec((_RB, DIM), lambda i: (i, 0)),
        pl.BlockSpec((1, DIM), lambda i: (0, 0)),
        pl.BlockSpec((1, DIM), lambda i: (0, 0)),
        pl.BlockSpec((DIM, 48), lambda i: (0, 0)),
        pl.BlockSpec((DIM, 48), lambda i: (0, 0)),
    ],
    out_specs=[
        pl.BlockSpec((_RB, 48), lambda i: (i, 0)),
        pl.BlockSpec((_RB, 48), lambda i: (i, 0)),
    ],
    out_shape=[
        jax.ShapeDtypeStruct((NPAD, 48), jnp.float32),
        jax.ShapeDtypeStruct((NPAD, 48), jnp.float32),
    ],
)


def _final_body(ra_ref, rb_ref, wlp_ref, b2c_ref, b2h_ref, blp_ref, o_ref):
    za = jnp.dot(ra_ref[:, :40], wlp_ref[:40],
                 preferred_element_type=jnp.float32)
    zb = jnp.dot(rb_ref[:, :40], wlp_ref[40:],
                 preferred_element_type=jnp.float32)
    bias = (jnp.dot(b2c_ref[...], wlp_ref[:40],
                    preferred_element_type=jnp.float32)
            + jnp.dot(b2h_ref[...], wlp_ref[40:],
                      preferred_element_type=jnp.float32)
            + blp_ref[...])
    z = za + zb + bias
    m = jnp.max(z, axis=1, keepdims=True)
    ez = jnp.exp(z - m)
    o_ref[...] = z - m - jnp.log(jnp.sum(ez, axis=1, keepdims=True))


_final = pl.pallas_call(
    _final_body,
    grid=(_GRID,),
    in_specs=[
        pl.BlockSpec((_RB, 48), lambda i: (i, 0)),
        pl.BlockSpec((_RB, 48), lambda i: (i, 0)),
        pl.BlockSpec((80, NC), lambda i: (0, 0)),
        pl.BlockSpec((1, NC), lambda i: (0, 0)),
        pl.BlockSpec((1, NC), lambda i: (0, 0)),
        pl.BlockSpec((1, NC), lambda i: (0, 0)),
    ],
    out_specs=pl.BlockSpec((_RB, NC), lambda i: (i, 0)),
    out_shape=jax.ShapeDtypeStruct((NPAD, NC), jnp.float32),
)


def kernel(x, edge_index, hyperedge_index, W1c, b1c, W1h, b1h,
           W2c, b2c, W2h, b2h, Wlp, blp):
    ni = hyperedge_index[0]
    ei = hyperedge_index[1]
    pad_n = jnp.full((NNZ_PAD,), N, jnp.int32)
    pad_e = jnp.full((NNZ_PAD,), NE, jnp.int32)
    ni_blk = jnp.concatenate([ni, pad_n]).reshape(NBLK, IDX_BLK)
    ei_blk = jnp.concatenate([ei, pad_e]).reshape(NBLK, IDX_BLK)

    x_pad = jnp.pad(x, ((0, NPAD - N), (0, 0)))
    w2c_pad = jnp.pad(W2c, ((0, 0), (0, 8)))
    w2h_pad = jnp.pad(W2h, ((0, 0), (0, 8)))

    dinv16, binv16 = _degree(ni_blk, ei_blk)

    xa, xb = _mm1(x_pad, W1c, W1h)
    _, _, p1a, p1b = _layer64(xa, xb, ni_blk, ei_blk, binv16, dinv16)
    qa, qb = _mid(p1a, p1b, b1c.reshape(1, DIM), b1h.reshape(1, DIM),
                  w2c_pad, w2h_pad)
    _, _, r2a, r2b = _layer48(qa, qb, ni_blk, ei_blk, binv16, dinv16)
    out = _final(r2a, r2b, Wlp, b2c.reshape(1, NC), b2h.reshape(1, NC),
                 blp.reshape(1, NC))
    return out[:N]


# speedup vs baseline: 1.6804x; 1.6804x over previous
import functools

import jax
import jax.numpy as jnp
from jax import lax
from jax.experimental import pallas as pl
from jax.experimental.pallas import tpu as pltpu
from jax.experimental.pallas import tpu_sc as plsc

N = 10000
NE = 10000
NNZ = 320000
DIN = 128
DIM = 64
NC = 40

NCORES = 2
NSUB = 16
NPAD = 10112
ROWS_PER_TILE = NPAD // NSUB
IDX_BLK = 128
DCHUNK = ROWS_PER_TILE // 4
NBLK = 2560
BLK_PER_TILE = NBLK // NSUB
NNZ_PAD = NBLK * IDX_BLK - NNZ

_sc_mesh = plsc.VectorSubcoreMesh(
    core_axis_name="c", subcore_axis_name="s",
    num_cores=NCORES, num_subcores=NSUB)
_sc_params = pltpu.CompilerParams(use_tc_tiling_on_sc=False)


def _degree_body(ni_hbm, ei_hbm, dinv_hbm, binv_hbm,
                 iv, ones_v, dbuf, acc_sh):
    c = lax.axis_index("c")
    s = lax.axis_index("s")
    base = s * ROWS_PER_TILE
    t0 = s * BLK_PER_TILE

    @pl.when(c == 0)
    def _():
        pltpu.sync_copy(ni_hbm.at[pl.ds(t0, BLK_PER_TILE)], iv)

    @pl.when(c == 1)
    def _():
        pltpu.sync_copy(ei_hbm.at[pl.ds(t0, BLK_PER_TILE)], iv)

    def fill(r, carry):
        ones_v[r, :] = jnp.ones((16,), jnp.float32)
        dbuf[r, :] = jnp.zeros((16,), jnp.float32)
        return carry

    lax.fori_loop(0, IDX_BLK, fill, 0)

    def zero(r, carry):
        dbuf[r, :] = jnp.zeros((16,), jnp.float32)
        return carry

    lax.fori_loop(IDX_BLK, ROWS_PER_TILE, zero, 0)
    pltpu.sync_copy(dbuf, acc_sh.at[pl.ds(base, ROWS_PER_TILE)])
    plsc.subcore_barrier()

    def step(b, carry):
        pltpu.sync_copy(ones_v, acc_sh.at[iv.at[b]], add=True)
        return carry

    lax.fori_loop(0, BLK_PER_TILE, step, 0)
    plsc.subcore_barrier()

    pltpu.sync_copy(acc_sh.at[pl.ds(base, ROWS_PER_TILE)], dbuf)

    def inv(r, carry):
        v = dbuf[r, :]
        dbuf[r, :] = jnp.where(v > 0.0, 1.0 / v, 0.0)
        return carry

    lax.fori_loop(0, ROWS_PER_TILE, inv, 0)

    @pl.when(c == 0)
    def _():
        pltpu.sync_copy(dbuf, dinv_hbm.at[pl.ds(base, ROWS_PER_TILE)])

    @pl.when(c == 1)
    def _():
        pltpu.sync_copy(dbuf, binv_hbm.at[pl.ds(base, ROWS_PER_TILE)])


_degree = pl.kernel(
    _degree_body,
    out_type=(jax.ShapeDtypeStruct((NPAD, 16), jnp.float32),
              jax.ShapeDtypeStruct((NPAD, 16), jnp.float32)),
    mesh=_sc_mesh,
    scratch_types=[
        pltpu.VMEM((BLK_PER_TILE, IDX_BLK), jnp.int32),
        pltpu.VMEM((IDX_BLK, 16), jnp.float32),
        pltpu.VMEM((ROWS_PER_TILE, 16), jnp.float32),
        pltpu.VMEM_SHARED((NPAD, 16), jnp.float32),
    ],
    compiler_params=_sc_params,
)


NSLOT = 4
GRP = 4
NGRP = BLK_PER_TILE // GRP
_DRAIN_CHUNKS = [(0, 128), (128, 128), (256, 128), (384, 128), (512, 120)]


def _layer_body(F, xa, xb, ni_hbm, ei_hbm, binv_hbm, dinv_hbm,
                pa, pb,
                wg, ws, rows, sbuf, tbl_sh, acc_sh, gsem, ssem, isem):
    c = lax.axis_index("c")
    s = lax.axis_index("s")
    base = s * ROWS_PER_TILE
    t0 = s * BLK_PER_TILE
    nk = F // 16

    def zero_acc():
        def zero(r, carry):
            for k in range(nk):
                rows[0, r, pl.ds(k * 16, 16)] = jnp.zeros((16,), jnp.float32)
            return carry

        lax.fori_loop(0, 128, zero, 0)
        for off, sz in _DRAIN_CHUNKS:
            pltpu.sync_copy(rows.at[0, pl.ds(0, sz)],
                            acc_sh.at[pl.ds(base + off, sz)])

    def accumulate(gsrc_hbm, ssrc_hbm):
        def prefetch(w, wslot):
            @pl.when(w < NGRP)
            def _():
                pltpu.async_copy(gsrc_hbm.at[pl.ds(t0 + w * GRP, GRP)],
                                 wg.at[wslot], isem)
                pltpu.async_copy(ssrc_hbm.at[pl.ds(t0 + w * GRP, GRP)],
                                 ws.at[wslot], isem)

        def wait_window(w, wslot):
            pltpu.make_async_copy(gsrc_hbm.at[pl.ds(t0 + w * GRP, GRP)],
                                  wg.at[wslot], isem).wait()
            pltpu.make_async_copy(ssrc_hbm.at[pl.ds(t0 + w * GRP, GRP)],
                                  ws.at[wslot], isem).wait()

        def gather(wslot, j, k):
            pltpu.async_copy(tbl_sh.at[wg.at[wslot, j]], rows.at[k], gsem)

        def wait_gather(wslot, j, k):
            pltpu.make_async_copy(tbl_sh.at[wg.at[wslot, j]],
                                  rows.at[k], gsem).wait()

        def scatter(wslot, j, k):
            return pltpu.async_copy(rows.at[k], acc_sh.at[ws.at[wslot, j]],
                                    ssem, add=True)

        pltpu.sync_copy(gsrc_hbm.at[pl.ds(t0, GRP)], wg.at[0])
        pltpu.sync_copy(ssrc_hbm.at[pl.ds(t0, GRP)], ws.at[0])
        prefetch(1, 1)
        for k in range(NSLOT):
            gather(0, k, k)

        def group(g, carry):
            wcur = lax.rem(g, 3)
            wnxt = lax.rem(g + 1, 3)
            prefetch(g + 2, lax.rem(g + 2, 3))
            descs = []
            for bank in range(NSLOT // 2):
                k0 = 2 * bank
                wait_gather(wcur, k0, k0)
                wait_gather(wcur, k0 + 1, k0 + 1)
                descs.append(scatter(wcur, k0, k0))
                descs.append(scatter(wcur, k0 + 1, k0 + 1))
            for d in descs:
                d.wait()

            @pl.when(g + 1 < NGRP)
            def _():
                wait_window(g + 1, wnxt)
                for k in range(NSLOT):
                    gather(wnxt, k, k)
            return carry

        lax.fori_loop(0, NGRP, group, 0)

    def drain(scale_hbm, to_hbm, dst_a=None, dst_b=None):
        for off, sz in _DRAIN_CHUNKS:
            pltpu.sync_copy(scale_hbm.at[pl.ds(base + off, sz)],
                            sbuf.at[pl.ds(0, sz)])
            pltpu.sync_copy(acc_sh.at[pl.ds(base + off, sz)],
                            rows.at[0, pl.ds(0, sz)])

            def scale(r, carry):
                sv = sbuf[r, :]
                for k in range(nk):
                    rows[0, r, pl.ds(k * 16, 16)] = (
                        rows[0, r, pl.ds(k * 16, 16)] * sv)
                return carry

            lax.fori_loop(0, sz, scale, 0)

            if to_hbm:
                @pl.when(c == 0)
                def _():
                    pltpu.sync_copy(rows.at[0, pl.ds(0, sz)],
                                    dst_a.at[pl.ds(base + off, sz)])

                @pl.when(c == 1)
                def _():
                    pltpu.sync_copy(rows.at[0, pl.ds(0, sz)],
                                    dst_b.at[pl.ds(base + off, sz)])
            else:
                pltpu.sync_copy(rows.at[0, pl.ds(0, sz)],
                                tbl_sh.at[pl.ds(base + off, sz)])

    @pl.when(c == 0)
    def _():
        pltpu.sync_copy(xa.at[pl.ds(base, ROWS_PER_TILE)],
                        tbl_sh.at[pl.ds(base, ROWS_PER_TILE)])

    @pl.when(c == 1)
    def _():
        pltpu.sync_copy(xb.at[pl.ds(base, ROWS_PER_TILE)],
                        tbl_sh.at[pl.ds(base, ROWS_PER_TILE)])

    zero_acc()
    plsc.subcore_barrier()
    accumulate(ni_hbm, ei_hbm)
    plsc.subcore_barrier()
    drain(binv_hbm, to_hbm=False)
    zero_acc()
    plsc.subcore_barrier()
    accumulate(ei_hbm, ni_hbm)
    plsc.subcore_barrier()
    drain(dinv_hbm, to_hbm=True, dst_a=pa, dst_b=pb)


def _make_layer(F):
    return pl.kernel(
        functools.partial(_layer_body, F),
        out_type=(jax.ShapeDtypeStruct((NPAD, F), jnp.float32),) * 2,
        mesh=_sc_mesh,
        scratch_types=[
            pltpu.VMEM((3, GRP, IDX_BLK), jnp.int32),
            pltpu.VMEM((3, GRP, IDX_BLK), jnp.int32),
            pltpu.VMEM((NSLOT, IDX_BLK, F), jnp.float32),
            pltpu.VMEM((IDX_BLK, 16), jnp.float32),
            pltpu.VMEM_SHARED((NPAD, F), jnp.float32),
            pltpu.VMEM_SHARED((NPAD, F), jnp.float32),
            pltpu.SemaphoreType.DMA,
            pltpu.SemaphoreType.DMA,
            pltpu.SemaphoreType.DMA,
        ],
        compiler_params=_sc_params,
    )


_layer64 = _make_layer(64)
_layer48 = _make_layer(48)


_RB = 1264
_GRID = NPAD // _RB


def _mm1_body(x_ref, wc_ref, wh_ref, oa_ref, ob_ref):
    xb = x_ref[...]
    oa_ref[...] = jnp.dot(xb, wc_ref[...], preferred_element_type=jnp.float32)
    ob_ref[...] = jnp.dot(xb, wh_ref[...], preferred_element_type=jnp.float32)


_mm1 = pl.pallas_call(
    _mm1_body,
    grid=(_GRID,),
    in_specs=[
        pl.BlockSpec((_RB, DIN), lambda i: (i, 0)),
        pl.BlockSpec((DIN, DIM), lambda i: (0, 0)),
        pl.BlockSpec((DIN, DIM), lambda i: (0, 0)),
    ],
    out_specs=[
        pl.BlockSpec((_RB, DIM), lambda i: (i, 0)),
        pl.BlockSpec((_RB, DIM), lambda i: (i, 0)),
    ],
    out_shape=[
        jax.ShapeDtypeStruct((NPAD, DIM), jnp.float32),
        jax.ShapeDtypeStruct((NPAD, DIM), jnp.float32),
    ],
)


def _mid_body(pa_ref, pb_ref, b1c_ref, b1h_ref, wc_ref, wh_ref,
              qa_ref, qb_ref):
    ga = jax.nn.relu(pa_ref[...] + b1c_ref[...])
    gb = jax.nn.relu(pb_ref[...] + b1h_ref[...])
    qa_ref[...] = jnp.dot(ga, wc_ref[...], preferred_element_type=jnp.float32)
    qb_ref[...] = jnp.dot(gb, wh_ref[...], preferred_element_type=jnp.float32)


_mid = pl.pallas_call(
    _mid_body,
    grid=(_GRID,),
    in_specs=[
        pl.BlockSpec((_RB, DIM), lambda i: (i, 0)),
        pl.BlockSpec((_RB, DIM), lambda i: (i, 0)),
        pl.BlockSpec((1, DIM), lambda i: (0, 0)),
        pl.BlockSpec((1, DIM), lambda i: (0, 0)),
        pl.BlockSpec((DIM, 48), lambda i: (0, 0)),
        pl.BlockSpec((DIM, 48), lambda i: (0, 0)),
    ],
    out_specs=[
        pl.BlockSpec((_RB, 48), lambda i: (i, 0)),
        pl.BlockSpec((_RB, 48), lambda i: (i, 0)),
    ],
    out_shape=[
        jax.ShapeDtypeStruct((NPAD, 48), jnp.float32),
        jax.ShapeDtypeStruct((NPAD, 48), jnp.float32),
    ],
)


def _final_body(ra_ref, rb_ref, wlp_ref, b2c_ref, b2h_ref, blp_ref, o_ref):
    za = jnp.dot(ra_ref[:, :40], wlp_ref[:40],
                 preferred_element_type=jnp.float32)
    zb = jnp.dot(rb_ref[:, :40], wlp_ref[40:],
                 preferred_element_type=jnp.float32)
    bias = (jnp.dot(b2c_ref[...], wlp_ref[:40],
                    preferred_element_type=jnp.float32)
            + jnp.dot(b2h_ref[...], wlp_ref[40:],
                      preferred_element_type=jnp.float32)
            + blp_ref[...])
    z = za + zb + bias
    m = jnp.max(z, axis=1, keepdims=True)
    ez = jnp.exp(z - m)
    o_ref[...] = z - m - jnp.log(jnp.sum(ez, axis=1, keepdims=True))


_final = pl.pallas_call(
    _final_body,
    grid=(_GRID,),
    in_specs=[
        pl.BlockSpec((_RB, 48), lambda i: (i, 0)),
        pl.BlockSpec((_RB, 48), lambda i: (i, 0)),
        pl.BlockSpec((80, NC), lambda i: (0, 0)),
        pl.BlockSpec((1, NC), lambda i: (0, 0)),
        pl.BlockSpec((1, NC), lambda i: (0, 0)),
        pl.BlockSpec((1, NC), lambda i: (0, 0)),
    ],
    out_specs=pl.BlockSpec((_RB, NC), lambda i: (i, 0)),
    out_shape=jax.ShapeDtypeStruct((NPAD, NC), jnp.float32),
)


def kernel(x, edge_index, hyperedge_index, W1c, b1c, W1h, b1h,
           W2c, b2c, W2h, b2h, Wlp, blp):
    ni = hyperedge_index[0]
    ei = hyperedge_index[1]
    pad_n = jnp.full((NNZ_PAD,), N, jnp.int32)
    pad_e = jnp.full((NNZ_PAD,), NE, jnp.int32)
    ni_blk = jnp.concatenate([ni, pad_n]).reshape(NBLK, IDX_BLK)
    ei_blk = jnp.concatenate([ei, pad_e]).reshape(NBLK, IDX_BLK)

    x_pad = jnp.pad(x, ((0, NPAD - N), (0, 0)))
    w2c_pad = jnp.pad(W2c, ((0, 0), (0, 8)))
    w2h_pad = jnp.pad(W2h, ((0, 0), (0, 8)))

    dinv16, binv16 = _degree(ni_blk, ei_blk)

    xa, xb = _mm1(x_pad, W1c, W1h)
    p1a, p1b = _layer64(xa, xb, ni_blk, ei_blk, binv16, dinv16)
    qa, qb = _mid(p1a, p1b, b1c.reshape(1, DIM), b1h.reshape(1, DIM),
                  w2c_pad, w2h_pad)
    r2a, r2b = _layer48(qa, qb, ni_blk, ei_blk, binv16, dinv16)
    out = _final(r2a, r2b, Wlp, b2c.reshape(1, NC), b2h.reshape(1, NC),
                 blp.reshape(1, NC))
    return out[:N]


# speedup vs baseline: 1.6948x; 1.0086x over previous
import functools

import jax
import jax.numpy as jnp
from jax import lax
from jax.experimental import pallas as pl
from jax.experimental.pallas import tpu as pltpu
from jax.experimental.pallas import tpu_sc as plsc

N = 10000
NE = 10000
NNZ = 320000
DIN = 128
DIM = 64
NC = 40

NCORES = 2
NSUB = 16
NPAD = 10112
ROWS_PER_TILE = NPAD // NSUB
IDX_BLK = 128
DCHUNK = ROWS_PER_TILE // 4
NBLK = 2560
BLK_PER_TILE = NBLK // NSUB
NNZ_PAD = NBLK * IDX_BLK - NNZ

_sc_mesh = plsc.VectorSubcoreMesh(
    core_axis_name="c", subcore_axis_name="s",
    num_cores=NCORES, num_subcores=NSUB)
_sc_params = pltpu.CompilerParams(use_tc_tiling_on_sc=False)


def _degree_body(ni_hbm, ei_hbm, dinv_hbm, binv_hbm,
                 iv, ones_v, dbuf, acc_sh):
    c = lax.axis_index("c")
    s = lax.axis_index("s")
    base = s * ROWS_PER_TILE
    t0 = s * BLK_PER_TILE

    @pl.when(c == 0)
    def _():
        pltpu.sync_copy(ni_hbm.at[pl.ds(t0, BLK_PER_TILE)], iv)

    @pl.when(c == 1)
    def _():
        pltpu.sync_copy(ei_hbm.at[pl.ds(t0, BLK_PER_TILE)], iv)

    def fill(r, carry):
        ones_v[r, :] = jnp.ones((16,), jnp.float32)
        dbuf[r, :] = jnp.zeros((16,), jnp.float32)
        return carry

    lax.fori_loop(0, IDX_BLK, fill, 0)

    def zero(r, carry):
        dbuf[r, :] = jnp.zeros((16,), jnp.float32)
        return carry

    lax.fori_loop(IDX_BLK, ROWS_PER_TILE, zero, 0)
    pltpu.sync_copy(dbuf, acc_sh.at[pl.ds(base, ROWS_PER_TILE)])
    plsc.subcore_barrier()

    def step(b, carry):
        pltpu.sync_copy(ones_v, acc_sh.at[iv.at[b]], add=True)
        return carry

    lax.fori_loop(0, BLK_PER_TILE, step, 0)
    plsc.subcore_barrier()

    pltpu.sync_copy(acc_sh.at[pl.ds(base, ROWS_PER_TILE)], dbuf)

    def inv(r, carry):
        v = dbuf[r, :]
        dbuf[r, :] = jnp.where(v > 0.0, 1.0 / v, 0.0)
        return carry

    lax.fori_loop(0, ROWS_PER_TILE, inv, 0)

    @pl.when(c == 0)
    def _():
        pltpu.sync_copy(dbuf, dinv_hbm.at[pl.ds(base, ROWS_PER_TILE)])

    @pl.when(c == 1)
    def _():
        pltpu.sync_copy(dbuf, binv_hbm.at[pl.ds(base, ROWS_PER_TILE)])


_degree = pl.kernel(
    _degree_body,
    out_type=(jax.ShapeDtypeStruct((NPAD, 16), jnp.float32),
              jax.ShapeDtypeStruct((NPAD, 16), jnp.float32)),
    mesh=_sc_mesh,
    scratch_types=[
        pltpu.VMEM((BLK_PER_TILE, IDX_BLK), jnp.int32),
        pltpu.VMEM((IDX_BLK, 16), jnp.float32),
        pltpu.VMEM((ROWS_PER_TILE, 16), jnp.float32),
        pltpu.VMEM_SHARED((NPAD, 16), jnp.float32),
    ],
    compiler_params=_sc_params,
)


_DRAIN_CHUNKS = [(0, 128), (128, 128), (256, 128), (384, 128), (512, 120)]


def _layer_body(F, NSLOT, xa, xb, ni_hbm, ei_hbm, binv_hbm, dinv_hbm,
                pa, pb,
                wg, ws, rows, sbuf, tbl_sh, acc_sh, gsem, ssem, isem):
    c = lax.axis_index("c")
    s = lax.axis_index("s")
    base = s * ROWS_PER_TILE
    t0 = s * BLK_PER_TILE
    nk = F // 16
    GRP = NSLOT
    NGRP = BLK_PER_TILE // GRP

    def zero_acc():
        def zero(r, carry):
            for k in range(nk):
                rows[0, r, pl.ds(k * 16, 16)] = jnp.zeros((16,), jnp.float32)
            return carry

        lax.fori_loop(0, 128, zero, 0)
        for off, sz in _DRAIN_CHUNKS:
            pltpu.sync_copy(rows.at[0, pl.ds(0, sz)],
                            acc_sh.at[pl.ds(base + off, sz)])

    def accumulate(gsrc_hbm, ssrc_hbm):
        def prefetch(w, wslot):
            @pl.when(w < NGRP)
            def _():
                pltpu.async_copy(gsrc_hbm.at[pl.ds(t0 + w * GRP, GRP)],
                                 wg.at[wslot], isem)
                pltpu.async_copy(ssrc_hbm.at[pl.ds(t0 + w * GRP, GRP)],
                                 ws.at[wslot], isem)

        def wait_window(w, wslot):
            pltpu.make_async_copy(gsrc_hbm.at[pl.ds(t0 + w * GRP, GRP)],
                                  wg.at[wslot], isem).wait()
            pltpu.make_async_copy(ssrc_hbm.at[pl.ds(t0 + w * GRP, GRP)],
                                  ws.at[wslot], isem).wait()

        def gather(wslot, j, k):
            pltpu.async_copy(tbl_sh.at[wg.at[wslot, j]], rows.at[k], gsem)

        def wait_gather(wslot, j, k):
            pltpu.make_async_copy(tbl_sh.at[wg.at[wslot, j]],
                                  rows.at[k], gsem).wait()

        def scatter(wslot, j, k):
            return pltpu.async_copy(rows.at[k], acc_sh.at[ws.at[wslot, j]],
                                    ssem, add=True)

        pltpu.sync_copy(gsrc_hbm.at[pl.ds(t0, GRP)], wg.at[0])
        pltpu.sync_copy(ssrc_hbm.at[pl.ds(t0, GRP)], ws.at[0])
        prefetch(1, 1)
        for k in range(NSLOT):
            gather(0, k, k)

        def group(g, carry):
            wcur = lax.rem(g, 3)
            wnxt = lax.rem(g + 1, 3)
            prefetch(g + 2, lax.rem(g + 2, 3))
            descs = []
            for bank in range(NSLOT // 2):
                k0 = 2 * bank
                wait_gather(wcur, k0, k0)
                wait_gather(wcur, k0 + 1, k0 + 1)
                descs.append(scatter(wcur, k0, k0))
                descs.append(scatter(wcur, k0 + 1, k0 + 1))
            for d in descs:
                d.wait()

            @pl.when(g + 1 < NGRP)
            def _():
                wait_window(g + 1, wnxt)
                for k in range(NSLOT):
                    gather(wnxt, k, k)
            return carry

        lax.fori_loop(0, NGRP, group, 0)

    def drain(scale_hbm, to_hbm, dst_a=None, dst_b=None):
        for off, sz in _DRAIN_CHUNKS:
            pltpu.sync_copy(scale_hbm.at[pl.ds(base + off, sz)],
                            sbuf.at[pl.ds(0, sz)])
            pltpu.sync_copy(acc_sh.at[pl.ds(base + off, sz)],
                            rows.at[0, pl.ds(0, sz)])

            def scale(r, carry):
                sv = sbuf[r, :]
                for k in range(nk):
                    rows[0, r, pl.ds(k * 16, 16)] = (
                        rows[0, r, pl.ds(k * 16, 16)] * sv)
                return carry

            lax.fori_loop(0, sz, scale, 0)

            if to_hbm:
                @pl.when(c == 0)
                def _():
                    pltpu.sync_copy(rows.at[0, pl.ds(0, sz)],
                                    dst_a.at[pl.ds(base + off, sz)])

                @pl.when(c == 1)
                def _():
                    pltpu.sync_copy(rows.at[0, pl.ds(0, sz)],
                                    dst_b.at[pl.ds(base + off, sz)])
            else:
                pltpu.sync_copy(rows.at[0, pl.ds(0, sz)],
                                tbl_sh.at[pl.ds(base + off, sz)])

    @pl.when(c == 0)
    def _():
        pltpu.sync_copy(xa.at[pl.ds(base, ROWS_PER_TILE)],
                        tbl_sh.at[pl.ds(base, ROWS_PER_TILE)])

    @pl.when(c == 1)
    def _():
        pltpu.sync_copy(xb.at[pl.ds(base, ROWS_PER_TILE)],
                        tbl_sh.at[pl.ds(base, ROWS_PER_TILE)])

    zero_acc()
    plsc.subcore_barrier()
    accumulate(ni_hbm, ei_hbm)
    plsc.subcore_barrier()
    drain(binv_hbm, to_hbm=False)
    zero_acc()
    plsc.subcore_barrier()
    accumulate(ei_hbm, ni_hbm)
    plsc.subcore_barrier()
    drain(dinv_hbm, to_hbm=True, dst_a=pa, dst_b=pb)


def _make_layer(F, NSLOT):
    GRP = NSLOT
    return pl.kernel(
        functools.partial(_layer_body, F, NSLOT),
        out_type=(jax.ShapeDtypeStruct((NPAD, F), jnp.float32),) * 2,
        mesh=_sc_mesh,
        scratch_types=[
            pltpu.VMEM((3, GRP, IDX_BLK), jnp.int32),
            pltpu.VMEM((3, GRP, IDX_BLK), jnp.int32),
            pltpu.VMEM((NSLOT, IDX_BLK, F), jnp.float32),
            pltpu.VMEM((IDX_BLK, 16), jnp.float32),
            pltpu.VMEM_SHARED((NPAD, F), jnp.float32),
            pltpu.VMEM_SHARED((NPAD, F), jnp.float32),
            pltpu.SemaphoreType.DMA,
            pltpu.SemaphoreType.DMA,
            pltpu.SemaphoreType.DMA,
        ],
        compiler_params=_sc_params,
    )


_layer64 = _make_layer(64, 4)
_layer48 = _make_layer(48, 8)


_RB = 1264
_GRID = NPAD // _RB


def _mm1_body(x_ref, wc_ref, wh_ref, oa_ref, ob_ref):
    xb = x_ref[...]
    oa_ref[...] = jnp.dot(xb, wc_ref[...], preferred_element_type=jnp.float32)
    ob_ref[...] = jnp.dot(xb, wh_ref[...], preferred_element_type=jnp.float32)


_mm1 = pl.pallas_call(
    _mm1_body,
    grid=(_GRID,),
    in_specs=[
        pl.BlockSpec((_RB, DIN), lambda i: (i, 0)),
        pl.BlockSpec((DIN, DIM), lambda i: (0, 0)),
        pl.BlockSpec((DIN, DIM), lambda i: (0, 0)),
    ],
    out_specs=[
        pl.BlockSpec((_RB, DIM), lambda i: (i, 0)),
        pl.BlockSpec((_RB, DIM), lambda i: (i, 0)),
    ],
    out_shape=[
        jax.ShapeDtypeStruct((NPAD, DIM), jnp.float32),
        jax.ShapeDtypeStruct((NPAD, DIM), jnp.float32),
    ],
)


def _mid_body(pa_ref, pb_ref, b1c_ref, b1h_ref, wc_ref, wh_ref,
              qa_ref, qb_ref):
    ga = jax.nn.relu(pa_ref[...] + b1c_ref[...])
    gb = jax.nn.relu(pb_ref[...] + b1h_ref[...])
    qa_ref[...] = jnp.dot(ga, wc_ref[...], preferred_element_type=jnp.float32)
    qb_ref[...] = jnp.dot(gb, wh_ref[...], preferred_element_type=jnp.float32)


_mid = pl.pallas_call(
    _mid_body,
    grid=(_GRID,),
    in_specs=[
        pl.BlockSpec((_RB, DIM), lambda i: (i, 0)),
        pl.BlockSpec((_RB, DIM), lambda i: (i, 0)),
        pl.BlockSpec((1, DIM), lambda i: (0, 0)),
        pl.BlockSpec((1, DIM), lambda i: (0, 0)),
        pl.BlockSpec((DIM, 48), lambda i: (0, 0)),
        pl.BlockSpec((DIM, 48), lambda i: (0, 0)),
    ],
    out_specs=[
        pl.BlockSpec((_RB, 48), lambda i: (i, 0)),
        pl.BlockSpec((_RB, 48), lambda i: (i, 0)),
    ],
    out_shape=[
        jax.ShapeDtypeStruct((NPAD, 48), jnp.float32),
        jax.ShapeDtypeStruct((NPAD, 48), jnp.float32),
    ],
)


def _final_body(ra_ref, rb_ref, wlp_ref, b2c_ref, b2h_ref, blp_ref, o_ref):
    za = jnp.dot(ra_ref[:, :40], wlp_ref[:40],
                 preferred_element_type=jnp.float32)
    zb = jnp.dot(rb_ref[:, :40], wlp_ref[40:],
                 preferred_element_type=jnp.float32)
    bias = (jnp.dot(b2c_ref[...], wlp_ref[:40],
                    preferred_element_type=jnp.float32)
            + jnp.dot(b2h_ref[...], wlp_ref[40:],
                      preferred_element_type=jnp.float32)
            + blp_ref[...])
    z = za + zb + bias
    m = jnp.max(z, axis=1, keepdims=True)
    ez = jnp.exp(z - m)
    o_ref[...] = z - m - jnp.log(jnp.sum(ez, axis=1, keepdims=True))


_final = pl.pallas_call(
    _final_body,
    grid=(_GRID,),
    in_specs=[
        pl.BlockSpec((_RB, 48), lambda i: (i, 0)),
        pl.BlockSpec((_RB, 48), lambda i: (i, 0)),
        pl.BlockSpec((80, NC), lambda i: (0, 0)),
        pl.BlockSpec((1, NC), lambda i: (0, 0)),
        pl.BlockSpec((1, NC), lambda i: (0, 0)),
        pl.BlockSpec((1, NC), lambda i: (0, 0)),
    ],
    out_specs=pl.BlockSpec((_RB, NC), lambda i: (i, 0)),
    out_shape=jax.ShapeDtypeStruct((NPAD, NC), jnp.float32),
)


def kernel(x, edge_index, hyperedge_index, W1c, b1c, W1h, b1h,
           W2c, b2c, W2h, b2h, Wlp, blp):
    ni = hyperedge_index[0]
    ei = hyperedge_index[1]
    pad_n = jnp.full((NNZ_PAD,), N, jnp.int32)
    pad_e = jnp.full((NNZ_PAD,), NE, jnp.int32)
    ni_blk = jnp.concatenate([ni, pad_n]).reshape(NBLK, IDX_BLK)
    ei_blk = jnp.concatenate([ei, pad_e]).reshape(NBLK, IDX_BLK)

    x_pad = jnp.pad(x, ((0, NPAD - N), (0, 0)))
    w2c_pad = jnp.pad(W2c, ((0, 0), (0, 8)))
    w2h_pad = jnp.pad(W2h, ((0, 0), (0, 8)))

    dinv16, binv16 = _degree(ni_blk, ei_blk)

    xa, xb = _mm1(x_pad, W1c, W1h)
    p1a, p1b = _layer64(xa, xb, ni_blk, ei_blk, binv16, dinv16)
    qa, qb = _mid(p1a, p1b, b1c.reshape(1, DIM), b1h.reshape(1, DIM),
                  w2c_pad, w2h_pad)
    r2a, r2b = _layer48(qa, qb, ni_blk, ei_blk, binv16, dinv16)
    out = _final(r2a, r2b, Wlp, b2c.reshape(1, NC), b2h.reshape(1, NC),
                 blp.reshape(1, NC))
    return out[:N]


# speedup vs baseline: 1.7095x; 1.0087x over previous
import functools

import jax
import jax.numpy as jnp
from jax import lax
from jax.experimental import pallas as pl
from jax.experimental.pallas import tpu as pltpu
from jax.experimental.pallas import tpu_sc as plsc

N = 10000
NE = 10000
NNZ = 320000
DIN = 128
DIM = 64
NC = 40

NCORES = 2
NSUB = 16
NPAD = 10112
ROWS_PER_TILE = NPAD // NSUB
IDX_BLK = 128
DCHUNK = ROWS_PER_TILE // 4
NBLK = 2560
BLK_PER_TILE = NBLK // NSUB
NNZ_PAD = NBLK * IDX_BLK - NNZ

_sc_mesh = plsc.VectorSubcoreMesh(
    core_axis_name="c", subcore_axis_name="s",
    num_cores=NCORES, num_subcores=NSUB)
_sc_params = pltpu.CompilerParams(use_tc_tiling_on_sc=False)


def _degree_body(ni_hbm, ei_hbm, dinv_hbm, binv_hbm,
                 iv, ones_v, dbuf, acc_sh, dsem):
    c = lax.axis_index("c")
    s = lax.axis_index("s")
    base = s * ROWS_PER_TILE
    t0 = s * BLK_PER_TILE

    @pl.when(c == 0)
    def _():
        pltpu.sync_copy(ni_hbm.at[pl.ds(t0, BLK_PER_TILE)], iv)

    @pl.when(c == 1)
    def _():
        pltpu.sync_copy(ei_hbm.at[pl.ds(t0, BLK_PER_TILE)], iv)

    def fill(r, carry):
        ones_v[r, :] = jnp.ones((16,), jnp.float32)
        dbuf[r, :] = jnp.zeros((16,), jnp.float32)
        return carry

    lax.fori_loop(0, IDX_BLK, fill, 0)

    def zero(r, carry):
        dbuf[r, :] = jnp.zeros((16,), jnp.float32)
        return carry

    lax.fori_loop(IDX_BLK, ROWS_PER_TILE, zero, 0)
    pltpu.sync_copy(dbuf, acc_sh.at[pl.ds(base, ROWS_PER_TILE)])
    plsc.subcore_barrier()

    def step(g, carry):
        descs = [pltpu.async_copy(ones_v, acc_sh.at[iv.at[g * 8 + k]],
                                  dsem, add=True)
                 for k in range(8)]
        for d in descs:
            d.wait()
        return carry

    lax.fori_loop(0, BLK_PER_TILE // 8, step, 0)
    plsc.subcore_barrier()

    pltpu.sync_copy(acc_sh.at[pl.ds(base, ROWS_PER_TILE)], dbuf)

    def inv(r, carry):
        v = dbuf[r, :]
        dbuf[r, :] = jnp.where(v > 0.0, 1.0 / v, 0.0)
        return carry

    lax.fori_loop(0, ROWS_PER_TILE, inv, 0)

    @pl.when(c == 0)
    def _():
        pltpu.sync_copy(dbuf, dinv_hbm.at[pl.ds(base, ROWS_PER_TILE)])

    @pl.when(c == 1)
    def _():
        pltpu.sync_copy(dbuf, binv_hbm.at[pl.ds(base, ROWS_PER_TILE)])


_degree = pl.kernel(
    _degree_body,
    out_type=(jax.ShapeDtypeStruct((NPAD, 16), jnp.float32),
              jax.ShapeDtypeStruct((NPAD, 16), jnp.float32)),
    mesh=_sc_mesh,
    scratch_types=[
        pltpu.VMEM((BLK_PER_TILE, IDX_BLK), jnp.int32),
        pltpu.VMEM((IDX_BLK, 16), jnp.float32),
        pltpu.VMEM((ROWS_PER_TILE, 16), jnp.float32),
        pltpu.VMEM_SHARED((NPAD, 16), jnp.float32),
        pltpu.SemaphoreType.DMA,
    ],
    compiler_params=_sc_params,
)


_DRAIN_CHUNKS = [(0, 128), (128, 128), (256, 128), (384, 128), (512, 120)]


def _layer_body(F, NSLOT, xa, xb, ni_hbm, ei_hbm, binv_hbm, dinv_hbm,
                pa, pb,
                wg, ws, rows, sbuf, tbl_sh, acc_sh, gsem, ssem, isem):
    c = lax.axis_index("c")
    s = lax.axis_index("s")
    base = s * ROWS_PER_TILE
    t0 = s * BLK_PER_TILE
    nk = F // 16
    GRP = NSLOT
    NGRP = BLK_PER_TILE // GRP

    def zero_acc():
        def zero(r, carry):
            for k in range(nk):
                rows[0, r, pl.ds(k * 16, 16)] = jnp.zeros((16,), jnp.float32)
            return carry

        lax.fori_loop(0, 128, zero, 0)
        for off, sz in _DRAIN_CHUNKS:
            pltpu.sync_copy(rows.at[0, pl.ds(0, sz)],
                            acc_sh.at[pl.ds(base + off, sz)])

    def accumulate(gsrc_hbm, ssrc_hbm):
        def prefetch(w, wslot):
            @pl.when(w < NGRP)
            def _():
                pltpu.async_copy(gsrc_hbm.at[pl.ds(t0 + w * GRP, GRP)],
                                 wg.at[wslot], isem)
                pltpu.async_copy(ssrc_hbm.at[pl.ds(t0 + w * GRP, GRP)],
                                 ws.at[wslot], isem)

        def wait_window(w, wslot):
            pltpu.make_async_copy(gsrc_hbm.at[pl.ds(t0 + w * GRP, GRP)],
                                  wg.at[wslot], isem).wait()
            pltpu.make_async_copy(ssrc_hbm.at[pl.ds(t0 + w * GRP, GRP)],
                                  ws.at[wslot], isem).wait()

        def gather(wslot, j, k):
            pltpu.async_copy(tbl_sh.at[wg.at[wslot, j]], rows.at[k], gsem)

        def wait_gather(wslot, j, k):
            pltpu.make_async_copy(tbl_sh.at[wg.at[wslot, j]],
                                  rows.at[k], gsem).wait()

        def scatter(wslot, j, k):
            return pltpu.async_copy(rows.at[k], acc_sh.at[ws.at[wslot, j]],
                                    ssem, add=True)

        pltpu.sync_copy(gsrc_hbm.at[pl.ds(t0, GRP)], wg.at[0])
        pltpu.sync_copy(ssrc_hbm.at[pl.ds(t0, GRP)], ws.at[0])
        prefetch(1, 1)
        for k in range(NSLOT):
            gather(0, k, k)

        def group(g, carry):
            wcur = lax.rem(g, 3)
            wnxt = lax.rem(g + 1, 3)
            prefetch(g + 2, lax.rem(g + 2, 3))
            descs = []
            for bank in range(NSLOT // 2):
                k0 = 2 * bank
                wait_gather(wcur, k0, k0)
                wait_gather(wcur, k0 + 1, k0 + 1)
                descs.append(scatter(wcur, k0, k0))
                descs.append(scatter(wcur, k0 + 1, k0 + 1))
            for d in descs:
                d.wait()

            @pl.when(g + 1 < NGRP)
            def _():
                wait_window(g + 1, wnxt)
                for k in range(NSLOT):
                    gather(wnxt, k, k)
            return carry

        lax.fori_loop(0, NGRP, group, 0)

    def drain(scale_hbm, to_hbm, dst_a=None, dst_b=None):
        for off, sz in _DRAIN_CHUNKS:
            pltpu.sync_copy(scale_hbm.at[pl.ds(base + off, sz)],
                            sbuf.at[pl.ds(0, sz)])
            pltpu.sync_copy(acc_sh.at[pl.ds(base + off, sz)],
                            rows.at[0, pl.ds(0, sz)])

            def scale(r, carry):
                sv = sbuf[r, :]
                for k in range(nk):
                    rows[0, r, pl.ds(k * 16, 16)] = (
                        rows[0, r, pl.ds(k * 16, 16)] * sv)
                return carry

            lax.fori_loop(0, sz, scale, 0)

            if to_hbm:
                @pl.when(c == 0)
                def _():
                    pltpu.sync_copy(rows.at[0, pl.ds(0, sz)],
                                    dst_a.at[pl.ds(base + off, sz)])

                @pl.when(c == 1)
                def _():
                    pltpu.sync_copy(rows.at[0, pl.ds(0, sz)],
                                    dst_b.at[pl.ds(base + off, sz)])
            else:
                pltpu.sync_copy(rows.at[0, pl.ds(0, sz)],
                                tbl_sh.at[pl.ds(base + off, sz)])

    @pl.when(c == 0)
    def _():
        pltpu.sync_copy(xa.at[pl.ds(base, ROWS_PER_TILE)],
                        tbl_sh.at[pl.ds(base, ROWS_PER_TILE)])

    @pl.when(c == 1)
    def _():
        pltpu.sync_copy(xb.at[pl.ds(base, ROWS_PER_TILE)],
                        tbl_sh.at[pl.ds(base, ROWS_PER_TILE)])

    zero_acc()
    plsc.subcore_barrier()
    accumulate(ni_hbm, ei_hbm)
    plsc.subcore_barrier()
    drain(binv_hbm, to_hbm=False)
    zero_acc()
    plsc.subcore_barrier()
    accumulate(ei_hbm, ni_hbm)
    plsc.subcore_barrier()
    drain(dinv_hbm, to_hbm=True, dst_a=pa, dst_b=pb)


def _make_layer(F, NSLOT):
    GRP = NSLOT
    return pl.kernel(
        functools.partial(_layer_body, F, NSLOT),
        out_type=(jax.ShapeDtypeStruct((NPAD, F), jnp.float32),) * 2,
        mesh=_sc_mesh,
        scratch_types=[
            pltpu.VMEM((3, GRP, IDX_BLK), jnp.int32),
            pltpu.VMEM((3, GRP, IDX_BLK), jnp.int32),
            pltpu.VMEM((NSLOT, IDX_BLK, F), jnp.float32),
            pltpu.VMEM((IDX_BLK, 16), jnp.float32),
            pltpu.VMEM_SHARED((NPAD, F), jnp.float32),
            pltpu.VMEM_SHARED((NPAD, F), jnp.float32),
            pltpu.SemaphoreType.DMA,
            pltpu.SemaphoreType.DMA,
            pltpu.SemaphoreType.DMA,
        ],
        compiler_params=_sc_params,
    )


_layer64 = _make_layer(64, 4)
_layer48 = _make_layer(48, 8)


_RB = 1264
_GRID = NPAD // _RB


def _mm1_body(x_ref, wc_ref, wh_ref, oa_ref, ob_ref):
    xb = x_ref[...]
    oa_ref[...] = jnp.dot(xb, wc_ref[...], preferred_element_type=jnp.float32)
    ob_ref[...] = jnp.dot(xb, wh_ref[...], preferred_element_type=jnp.float32)


_mm1 = pl.pallas_call(
    _mm1_body,
    grid=(_GRID,),
    in_specs=[
        pl.BlockSpec((_RB, DIN), lambda i: (i, 0)),
        pl.BlockSpec((DIN, DIM), lambda i: (0, 0)),
        pl.BlockSpec((DIN, DIM), lambda i: (0, 0)),
    ],
    out_specs=[
        pl.BlockSpec((_RB, DIM), lambda i: (i, 0)),
        pl.BlockSpec((_RB, DIM), lambda i: (i, 0)),
    ],
    out_shape=[
        jax.ShapeDtypeStruct((NPAD, DIM), jnp.float32),
        jax.ShapeDtypeStruct((NPAD, DIM), jnp.float32),
    ],
)


def _mid_body(pa_ref, pb_ref, b1c_ref, b1h_ref, wc_ref, wh_ref,
              qa_ref, qb_ref):
    ga = jax.nn.relu(pa_ref[...] + b1c_ref[...])
    gb = jax.nn.relu(pb_ref[...] + b1h_ref[...])
    qa_ref[...] = jnp.dot(ga, wc_ref[...], preferred_element_type=jnp.float32)
    qb_ref[...] = jnp.dot(gb, wh_ref[...], preferred_element_type=jnp.float32)


_mid = pl.pallas_call(
    _mid_body,
    grid=(_GRID,),
    in_specs=[
        pl.BlockSpec((_RB, DIM), lambda i: (i, 0)),
        pl.BlockSpec((_RB, DIM), lambda i: (i, 0)),
        pl.BlockSpec((1, DIM), lambda i: (0, 0)),
        pl.BlockSpec((1, DIM), lambda i: (0, 0)),
        pl.BlockSpec((DIM, 48), lambda i: (0, 0)),
        pl.BlockSpec((DIM, 48), lambda i: (0, 0)),
    ],
    out_specs=[
        pl.BlockSpec((_RB, 48), lambda i: (i, 0)),
        pl.BlockSpec((_RB, 48), lambda i: (i, 0)),
    ],
    out_shape=[
        jax.ShapeDtypeStruct((NPAD, 48), jnp.float32),
        jax.ShapeDtypeStruct((NPAD, 48), jnp.float32),
    ],
)


def _final_body(ra_ref, rb_ref, wlp_ref, b2c_ref, b2h_ref, blp_ref, o_ref):
    za = jnp.dot(ra_ref[:, :40], wlp_ref[:40],
                 preferred_element_type=jnp.float32)
    zb = jnp.dot(rb_ref[:, :40], wlp_ref[40:],
                 preferred_element_type=jnp.float32)
    bias = (jnp.dot(b2c_ref[...], wlp_ref[:40],
                    preferred_element_type=jnp.float32)
            + jnp.dot(b2h_ref[...], wlp_ref[40:],
                      preferred_element_type=jnp.float32)
            + blp_ref[...])
    z = za + zb + bias
    m = jnp.max(z, axis=1, keepdims=True)
    ez = jnp.exp(z - m)
    o_ref[...] = z - m - jnp.log(jnp.sum(ez, axis=1, keepdims=True))


_final = pl.pallas_call(
    _final_body,
    grid=(_GRID,),
    in_specs=[
        pl.BlockSpec((_RB, 48), lambda i: (i, 0)),
        pl.BlockSpec((_RB, 48), lambda i: (i, 0)),
        pl.BlockSpec((80, NC), lambda i: (0, 0)),
        pl.BlockSpec((1, NC), lambda i: (0, 0)),
        pl.BlockSpec((1, NC), lambda i: (0, 0)),
        pl.BlockSpec((1, NC), lambda i: (0, 0)),
    ],
    out_specs=pl.BlockSpec((_RB, NC), lambda i: (i, 0)),
    out_shape=jax.ShapeDtypeStruct((NPAD, NC), jnp.float32),
)


def kernel(x, edge_index, hyperedge_index, W1c, b1c, W1h, b1h,
           W2c, b2c, W2h, b2h, Wlp, blp):
    ni = hyperedge_index[0]
    ei = hyperedge_index[1]
    pad_n = jnp.full((NNZ_PAD,), N, jnp.int32)
    pad_e = jnp.full((NNZ_PAD,), NE, jnp.int32)
    ni_blk = jnp.concatenate([ni, pad_n]).reshape(NBLK, IDX_BLK)
    ei_blk = jnp.concatenate([ei, pad_e]).reshape(NBLK, IDX_BLK)

    x_pad = jnp.pad(x, ((0, NPAD - N), (0, 0)))
    w2c_pad = jnp.pad(W2c, ((0, 0), (0, 8)))
    w2h_pad = jnp.pad(W2h, ((0, 0), (0, 8)))

    dinv16, binv16 = _degree(ni_blk, ei_blk)

    xa, xb = _mm1(x_pad, W1c, W1h)
    p1a, p1b = _layer64(xa, xb, ni_blk, ei_blk, binv16, dinv16)
    qa, qb = _mid(p1a, p1b, b1c.reshape(1, DIM), b1h.reshape(1, DIM),
                  w2c_pad, w2h_pad)
    r2a, r2b = _layer48(qa, qb, ni_blk, ei_blk, binv16, dinv16)
    out = _final(r2a, r2b, Wlp, b2c.reshape(1, NC), b2h.reshape(1, NC),
                 blp.reshape(1, NC))
    return out[:N]
